# baseline (speedup 1.0000x reference)
import functools

import jax
import jax.numpy as jnp
from jax import lax
from jax.experimental import pallas as pl
from jax.experimental.pallas import tpu as pltpu
from jax.experimental.pallas import tpu_sc as plsc

_NC = 2
_NS = 16
_NW = _NC * _NS
_LANES = 16
_IDX_CHUNK = 100


def _gather_pool_kernel(batch, hist, dim):
    n_chunk = hist // _IDX_CHUNK
    b_per_w = batch // _NW
    vregs = dim // _LANES
    mesh = plsc.VectorSubcoreMesh(core_axis_name="c", subcore_axis_name="s")

    unroll = 8

    @functools.partial(
        pl.kernel,
        mesh=mesh,
        out_type=jax.ShapeDtypeStruct((batch, dim), jnp.float32),
        scratch_types=[
            pltpu.VMEM((b_per_w, n_chunk, _IDX_CHUNK), jnp.int32),
            pltpu.VMEM((hist, dim), jnp.bfloat16),
            pltpu.VMEM((hist, dim), jnp.bfloat16),
            pltpu.VMEM((b_per_w, dim), jnp.float32),
            pltpu.SemaphoreType.DMA,
            pltpu.SemaphoreType.DMA,
        ],
        compiler_params=pltpu.CompilerParams(
            use_tc_tiling_on_sc=False, needs_layout_passes=False
        ),
    )
    def k(ids_hbm, emb_hbm, out_hbm, idx_v, rows0, rows1, out_v, sem0, sem1):
        wid = lax.axis_index("s") * _NC + lax.axis_index("c")
        base = wid * b_per_w

        def gather(s, rows, sem):
            for c in range(n_chunk):
                pltpu.async_copy(
                    emb_hbm.at[idx_v.at[s].at[c]],
                    rows.at[pl.ds(c * _IDX_CHUNK, _IDX_CHUNK)],
                    sem,
                )

        def drain(rows, sem):
            for c in range(n_chunk):
                pltpu.make_async_copy(
                    emb_hbm.at[idx_v.at[0].at[c]],
                    rows.at[pl.ds(c * _IDX_CHUNK, _IDX_CHUNK)],
                    sem,
                ).wait()

        def accum(rows, s_out):
            def acc_body(j, carry):
                new = carry
                for u in range(unroll):
                    cur = []
                    for g in range(vregs // 2):
                        packed = rows[j * unroll + u, pl.ds(2 * g * _LANES, 2 * _LANES)]
                        a, b = plsc.unpack(packed, format=plsc.PackFormat.INTERLEAVED)
                        cur.extend((a, b))
                    new = tuple(new[v] + cur[v] for v in range(vregs))
                return new

            acc = lax.fori_loop(
                0, hist // unroll, acc_body,
                tuple(jnp.zeros((_LANES,), jnp.float32) for _ in range(vregs)),
            )
            for v in range(vregs):
                out_v[s_out, pl.ds(v * _LANES, _LANES)] = acc[v]

        pltpu.sync_copy(ids_hbm.at[pl.ds(base, b_per_w)], idx_v)
        gather(0, rows0, sem0)

        def step(t, _):
            sa = 2 * t
            sb = 2 * t + 1
            gather(sb, rows1, sem1)
            drain(rows0, sem0)
            accum(rows0, sa)
            gather(jnp.minimum(sa + 2, b_per_w - 1), rows0, sem0)
            drain(rows1, sem1)
            accum(rows1, sb)
            return 0

        lax.fori_loop(0, b_per_w // 2, step, 0)
        drain(rows0, sem0)
        pltpu.sync_copy(out_v, out_hbm.at[pl.ds(base, b_per_w)])

    return k


def _mlp_body(inv_hist, sums_ref, w1_ref, b1_ref, w2_ref, b2_ref, out_ref):
    pooled = sums_ref[...] * inv_hist
    h = jnp.tanh(
        jnp.dot(pooled, w1_ref[...], preferred_element_type=jnp.float32)
        + b1_ref[...]
    )
    out_ref[...] = (
        jnp.dot(h, w2_ref[...], preferred_element_type=jnp.float32) + b2_ref[...]
    )


def kernel(input_ids, embedding, W1, b1, W2, b2):
    batch, hist = input_ids.shape
    _, dim = embedding.shape
    n_chunk = hist // _IDX_CHUNK

    ids = input_ids.astype(jnp.int32).reshape(batch, n_chunk, _IDX_CHUNK)
    sums = _gather_pool_kernel(batch, hist, dim)(
        ids, embedding.astype(jnp.bfloat16)
    )
    perm = jnp.array(
        [32 * g + 2 * k + p
         for g in range(dim // 32) for p in (0, 1) for k in range(16)],
        dtype=jnp.int32,
    )
    out = pl.pallas_call(
        functools.partial(_mlp_body, 1.0 / hist),
        out_shape=jax.ShapeDtypeStruct((batch, 1), jnp.float32),
    )(sums, W1[perm, :], b1.reshape(1, -1), W2, b2.reshape(1, 1))
    return out[:, 0]


# speedup vs baseline: 1.2566x; 1.2566x over previous
import functools

import jax
import jax.numpy as jnp
from jax import lax
from jax.experimental import pallas as pl
from jax.experimental.pallas import tpu as pltpu
from jax.experimental.pallas import tpu_sc as plsc

_NC = 2
_NS = 16
_NW = _NC * _NS
_LANES = 16
_IDX_CHUNK = 100


def _gather_pool_kernel(batch, hist, dim):
    n_chunk = hist // _IDX_CHUNK
    b_per_w = batch // _NW
    vregs = dim // _LANES
    mesh = plsc.VectorSubcoreMesh(core_axis_name="c", subcore_axis_name="s")

    unroll = 8

    @functools.partial(
        pl.kernel,
        mesh=mesh,
        out_type=jax.ShapeDtypeStruct((batch, dim), jnp.float32),
        scratch_types=[
            pltpu.VMEM((b_per_w, n_chunk, _IDX_CHUNK), jnp.int32),
            pltpu.VMEM((hist, dim), jnp.float32),
            pltpu.VMEM((hist, dim), jnp.float32),
            pltpu.VMEM((b_per_w, dim), jnp.float32),
            pltpu.SemaphoreType.DMA,
            pltpu.SemaphoreType.DMA,
        ],
        compiler_params=pltpu.CompilerParams(use_tc_tiling_on_sc=False),
    )
    def k(ids_hbm, emb_hbm, out_hbm, idx_v, rows0, rows1, out_v, sem0, sem1):
        wid = lax.axis_index("s") * _NC + lax.axis_index("c")
        base = wid * b_per_w

        def gather(s, rows, sem):
            for c in range(n_chunk):
                pltpu.async_copy(
                    emb_hbm.at[idx_v.at[s].at[c]],
                    rows.at[pl.ds(c * _IDX_CHUNK, _IDX_CHUNK)],
                    sem,
                )

        def drain(rows, sem):
            for c in range(n_chunk):
                pltpu.make_async_copy(
                    emb_hbm.at[idx_v.at[0].at[c]],
                    rows.at[pl.ds(c * _IDX_CHUNK, _IDX_CHUNK)],
                    sem,
                ).wait()

        def accum(rows, s_out):
            def acc_body(j, carry):
                new = carry
                for u in range(unroll):
                    new = tuple(
                        new[v] + rows[j * unroll + u, pl.ds(v * _LANES, _LANES)]
                        for v in range(vregs)
                    )
                return new

            acc = lax.fori_loop(
                0, hist // unroll, acc_body,
                tuple(jnp.zeros((_LANES,), jnp.float32) for _ in range(vregs)),
            )
            for v in range(vregs):
                out_v[s_out, pl.ds(v * _LANES, _LANES)] = acc[v]

        pltpu.sync_copy(ids_hbm.at[pl.ds(base, b_per_w)], idx_v)
        gather(0, rows0, sem0)

        def step(t, _):
            sa = 2 * t
            sb = 2 * t + 1
            gather(sb, rows1, sem1)
            drain(rows0, sem0)
            accum(rows0, sa)
            gather(jnp.minimum(sa + 2, b_per_w - 1), rows0, sem0)
            drain(rows1, sem1)
            accum(rows1, sb)
            return 0

        lax.fori_loop(0, b_per_w // 2, step, 0)
        drain(rows0, sem0)
        pltpu.sync_copy(out_v, out_hbm.at[pl.ds(base, b_per_w)])

    return k


def _mlp_body(inv_hist, sums_ref, w1_ref, b1_ref, w2_ref, b2_ref, out_ref):
    pooled = sums_ref[...] * inv_hist
    h = jnp.tanh(
        jnp.dot(pooled, w1_ref[...], preferred_element_type=jnp.float32)
        + b1_ref[...]
    )
    out_ref[...] = (
        jnp.dot(h, w2_ref[...], preferred_element_type=jnp.float32) + b2_ref[...]
    )


def kernel(input_ids, embedding, W1, b1, W2, b2):
    batch, hist = input_ids.shape
    _, dim = embedding.shape
    n_chunk = hist // _IDX_CHUNK

    ids = input_ids.astype(jnp.int32).reshape(batch, n_chunk, _IDX_CHUNK)
    sums = _gather_pool_kernel(batch, hist, dim)(ids, embedding)

    out = pl.pallas_call(
        functools.partial(_mlp_body, 1.0 / hist),
        out_shape=jax.ShapeDtypeStruct((batch, 1), jnp.float32),
    )(sums, W1, b1.reshape(1, -1), W2, b2.reshape(1, 1))
    return out[:, 0]
